# baseline (speedup 1.0000x reference)
import functools

import jax
import jax.numpy as jnp
from jax import lax
from jax.experimental import pallas as pl
from jax.experimental.pallas import tpu as pltpu
from jax.experimental.pallas import tpu_sc as plsc

NC = 2
NS = 16
CB = 128

_MAGIC = 0x5F3759DF


def _quake_rsqrt(v):
    ib = plsc.bitcast(v, jnp.int32)
    y = plsc.bitcast(jnp.full((16,), _MAGIC, jnp.int32)
                     - lax.shift_right_logical(ib, 1), jnp.float32)
    for _ in range(3):
        y = y * (1.5 - 0.5 * v * y * y)
    return y


def _ring_propagate(gh, src_v, dst_v, rows_v, acc, gsem, ssem, k_chunks, nb):
    for b in range(nb):
        pltpu.async_copy(gh.at[src_v.at[b]], rows_v.at[b], gsem)

    def body(j, carry):
        bj = lax.rem(j, nb)
        pltpu.make_async_copy(gh.at[src_v.at[bj]], rows_v.at[bj], gsem).wait()
        pltpu.async_copy(rows_v.at[bj], acc.at[dst_v.at[j]], ssem, add=True)
        nxt = j + (nb - 1)

        @pl.when((j >= 1) & (nxt < k_chunks))
        def _():
            bp = lax.rem(nxt, nb)
            pltpu.make_async_copy(rows_v.at[bp],
                                  acc.at[dst_v.at[0]], ssem).wait()
            pltpu.async_copy(gh.at[src_v.at[nxt]], rows_v.at[bp], gsem)

        return carry

    lax.fori_loop(0, k_chunks, body, 0)
    for _ in range(nb):
        pltpu.make_async_copy(rows_v.at[0], acc.at[dst_v.at[0]], ssem).wait()


def _make_fused_l1(np_rows, dh, k_chunks, n_sub, nb):
    rpt = np_rows // NS
    sub = rpt // n_sub
    mesh = plsc.VectorSubcoreMesh(core_axis_name="c", subcore_axis_name="s")

    @functools.partial(
        pl.kernel,
        out_type=[
            jax.ShapeDtypeStruct((NC, np_rows, dh), jnp.float32),
            jax.ShapeDtypeStruct((NC, np_rows, dh), jnp.float32),
            jax.ShapeDtypeStruct((np_rows, 16), jnp.float32),
        ],
        mesh=mesh,
        compiler_params=pltpu.CompilerParams(use_tc_tiling_on_sc=False,
                                             needs_layout_passes=False),
        scratch_types=[
            pltpu.VMEM((k_chunks, CB), jnp.int32),
            pltpu.VMEM((k_chunks, CB), jnp.int32),
            pltpu.VMEM((CB, 16), jnp.float32),
            pltpu.VMEM((nb, CB, dh), jnp.float32),
            pltpu.VMEM((sub, dh), jnp.float32),
            pltpu.VMEM((sub, 16), jnp.float32),
            pltpu.VMEM_SHARED((np_rows, 16), jnp.float32),
            pltpu.VMEM_SHARED((np_rows, dh), jnp.float32),
            pltpu.SemaphoreType.DMA,
            pltpu.SemaphoreType.DMA,
        ],
    )
    def fused_kernel(src_hbm, dst_hbm, xs_hbm, zeros16_hbm, zerosd_hbm,
                     ones_hbm, s1_hbm, g1_hbm, dinv_hbm,
                     src_v, dst_v, ones_v, rows_v, x_v, deg_v,
                     acc16, accd, gsem, ssem):
        c = lax.axis_index("c")
        s = lax.axis_index("s")
        pltpu.sync_copy(src_hbm.at[s], src_v)
        pltpu.sync_copy(dst_hbm.at[s], dst_v)
        pltpu.sync_copy(ones_hbm, ones_v)
        pltpu.sync_copy(zeros16_hbm.at[pl.ds(s * rpt, rpt)],
                        acc16.at[pl.ds(s * rpt, rpt)])
        pltpu.sync_copy(zerosd_hbm.at[pl.ds(s * rpt, rpt)],
                        accd.at[pl.ds(s * rpt, rpt)])
        plsc.subcore_barrier()

        def dbody(j, carry):
            pltpu.sync_copy(ones_v, acc16.at[dst_v.at[j]], add=True)
            return carry

        lax.fori_loop(0, k_chunks, dbody, 0)
        plsc.subcore_barrier()

        for k2 in range(n_sub):
            r0 = s * rpt + k2 * sub
            pltpu.sync_copy(xs_hbm.at[c, pl.ds(r0, sub)], x_v)
            pltpu.sync_copy(acc16.at[pl.ds(r0, sub)], deg_v)

            def rowfn(i, carry):
                y = _quake_rsqrt(deg_v[i, :])
                deg_v[i, :] = y
                for kk in range(dh // 16):
                    x_v[i, pl.ds(kk * 16, 16)] = x_v[i, pl.ds(kk * 16, 16)] * y
                return carry

            lax.fori_loop(0, sub, rowfn, 0)
            pltpu.sync_copy(x_v, g1_hbm.at[c, pl.ds(r0, sub)])

            @pl.when(c == 0)
            def _():
                pltpu.sync_copy(deg_v, dinv_hbm.at[pl.ds(r0, sub)])

        plsc.subcore_barrier()

        _ring_propagate(g1_hbm.at[c], src_v, dst_v, rows_v, accd,
                        gsem, ssem, k_chunks, nb)
        plsc.subcore_barrier()
        pltpu.sync_copy(accd.at[pl.ds(s * rpt, rpt)],
                        s1_hbm.at[c, pl.ds(s * rpt, rpt)])

    return fused_kernel


def _make_prop_final(np_rows, dh, k_chunks, nb):
    rpt = np_rows // NS
    mesh = plsc.VectorSubcoreMesh(core_axis_name="c", subcore_axis_name="s")

    @functools.partial(
        pl.kernel,
        out_type=jax.ShapeDtypeStruct((NC, np_rows, dh), jnp.float32),
        mesh=mesh,
        compiler_params=pltpu.CompilerParams(use_tc_tiling_on_sc=False),
        scratch_types=[
            pltpu.VMEM((k_chunks, CB), jnp.int32),
            pltpu.VMEM((k_chunks, CB), jnp.int32),
            pltpu.VMEM((nb, CB, dh), jnp.float32),
            pltpu.VMEM((np_rows // NS, dh), jnp.float32),
            pltpu.VMEM((np_rows // NS, 16), jnp.float32),
            pltpu.VMEM((dh,), jnp.float32),
            pltpu.VMEM_SHARED((np_rows, dh), jnp.float32),
            pltpu.SemaphoreType.DMA,
            pltpu.SemaphoreType.DMA,
        ],
    )
    def prop_kernel(src_hbm, dst_hbm, g_hbm, zeros_hbm, dinv_hbm, bias_hbm,
                    out_hbm, src_v, dst_v, rows_v, res_v, dinv_v, bias_v,
                    acc, gsem, ssem):
        c = lax.axis_index("c")
        s = lax.axis_index("s")
        pltpu.sync_copy(zeros_hbm.at[pl.ds(s * rpt, rpt)],
                        acc.at[pl.ds(s * rpt, rpt)])
        pltpu.sync_copy(src_hbm.at[s], src_v)
        pltpu.sync_copy(dst_hbm.at[s], dst_v)
        pltpu.sync_copy(dinv_hbm.at[pl.ds(s * rpt, rpt)], dinv_v)
        pltpu.sync_copy(bias_hbm.at[c], bias_v)
        plsc.subcore_barrier()

        _ring_propagate(g_hbm.at[c], src_v, dst_v, rows_v, acc,
                        gsem, ssem, k_chunks, nb)
        plsc.subcore_barrier()

        pltpu.sync_copy(acc.at[pl.ds(s * rpt, rpt)], res_v)
        bias = bias_v[:]

        def fin(i, carry):
            res_v[i, :] = res_v[i, :] * dinv_v[i, :] + bias
            return carry

        lax.fori_loop(0, rpt, fin, 0)
        pltpu.sync_copy(res_v, out_hbm.at[c, pl.ds(s * rpt, rpt)])

    return prop_kernel


def _mlp_body(dinv16_ref, s1_ref, w1_ref, b1_ref, w2_ref, g2_ref):
    dinv = dinv16_ref[:, 0:1]
    a1 = jnp.concatenate([s1_ref[0], s1_ref[1]], axis=1) * dinv
    h = jnp.dot(a1, w1_ref[...], preferred_element_type=jnp.float32)
    h = jnp.maximum(h + b1_ref[...], 0.0)
    t = jnp.dot(h, w2_ref[...], preferred_element_type=jnp.float32)
    g2 = t * dinv
    ch = t.shape[1] // 2
    g2_ref[0] = g2[:, :ch]
    g2_ref[1] = g2[:, ch:]


def kernel(x, edge_index, W1, b1, W2, b2):
    n, d = x.shape
    h_dim = W1.shape[1]
    c_dim = W2.shape[1]
    e = edge_index.shape[1]
    dh = d // 2

    loops = jnp.arange(n, dtype=jnp.int32)
    e_all = e + n
    k_prop = -(-e_all // (NS * CB))
    e_pad = NS * k_prop * CB
    pad = e_pad - e_all
    src = jnp.concatenate([edge_index[0], loops, jnp.zeros((pad,), jnp.int32)])
    dst = jnp.concatenate([edge_index[1], loops, jnp.full((pad,), n, jnp.int32)])
    src_p = src.reshape(NS, k_prop, CB)
    dst_p = dst.reshape(NS, k_prop, CB)

    np_rows = -(-(n + 1) // (NS * 8)) * NS * 8

    z16 = jnp.zeros((np_rows, 16), jnp.float32)
    zd = jnp.zeros((np_rows, dh), jnp.float32)
    zc = jnp.zeros((np_rows, c_dim // 2), jnp.float32)
    ones = jnp.ones((CB, 16), jnp.float32)
    xs = jnp.pad(x, ((0, np_rows - n), (0, 0))).reshape(np_rows, NC, dh)
    xs = jnp.transpose(xs, (1, 0, 2))

    s1f, _, dinv16 = _make_fused_l1(np_rows, dh, k_prop, 8, 3)(
        src_p, dst_p, xs, z16, zd, ones)
    s1 = s1f[:, :n]

    bn = 1000
    grid = (n // bn,)
    g2 = pl.pallas_call(
        _mlp_body,
        grid=grid,
        in_specs=[
            pl.BlockSpec((bn, 16), lambda i: (i, 0)),
            pl.BlockSpec((2, bn, dh), lambda i: (0, i, 0)),
            pl.BlockSpec((d, h_dim), lambda i: (0, 0)),
            pl.BlockSpec((1, h_dim), lambda i: (0, 0)),
            pl.BlockSpec((h_dim, c_dim), lambda i: (0, 0)),
        ],
        out_specs=pl.BlockSpec((2, bn, c_dim // 2), lambda i: (0, i, 0)),
        out_shape=jax.ShapeDtypeStruct((2, n, c_dim // 2), jnp.float32),
    )(dinv16[:n], s1, W1, b1.reshape(1, h_dim), W2)

    b2h = b2.reshape(NC, c_dim // 2)
    outh = _make_prop_final(np_rows, c_dim // 2, k_prop, 4)(
        src_p, dst_p, g2, zc, dinv16, b2h)
    return jnp.concatenate([outh[0, :n], outh[1, :n]], axis=1)


# speedup vs baseline: 1.3960x; 1.3960x over previous
import functools

import jax
import jax.numpy as jnp
from jax import lax
from jax.experimental import pallas as pl
from jax.experimental.pallas import tpu as pltpu
from jax.experimental.pallas import tpu_sc as plsc

NC = 2
NS = 16
NW = NC * NS
CB = 128


def _make_deg(np_rows, e):
    et = e // NW
    kf = et // CB
    tail = et - kf * CB
    rpt = np_rows // NS
    mesh = plsc.VectorSubcoreMesh(core_axis_name="c", subcore_axis_name="s")

    @functools.partial(
        pl.kernel,
        out_type=jax.ShapeDtypeStruct((NC, np_rows, 16), jnp.float32),
        mesh=mesh,
        compiler_params=pltpu.CompilerParams(use_tc_tiling_on_sc=False),
        scratch_types=[
            pltpu.VMEM((et,), jnp.int32),
            pltpu.VMEM((CB, 16), jnp.float32),
            pltpu.VMEM_SHARED((np_rows, 16), jnp.float32),
        ],
    )
    def deg_kernel(ei_hbm, zeros_hbm, ones_hbm, out_hbm, dst_v, ones_v, acc):
        c = lax.axis_index("c")
        s = lax.axis_index("s")
        wid = c * NS + s
        pltpu.sync_copy(zeros_hbm.at[pl.ds(s * rpt, rpt)],
                        acc.at[pl.ds(s * rpt, rpt)])
        pltpu.sync_copy(ei_hbm.at[1, pl.ds(wid * et, et)], dst_v)
        pltpu.sync_copy(ones_hbm, ones_v)
        plsc.subcore_barrier()

        def body(j, carry):
            pltpu.sync_copy(ones_v, acc.at[dst_v.at[pl.ds(j * CB, CB)]],
                            add=True)
            return carry

        lax.fori_loop(0, kf, body, 0)
        if tail:
            pltpu.sync_copy(ones_v.at[pl.ds(0, tail)],
                            acc.at[dst_v.at[pl.ds(kf * CB, tail)]], add=True)
        plsc.subcore_barrier()
        pltpu.sync_copy(acc.at[pl.ds(s * rpt, rpt)],
                        out_hbm.at[c, pl.ds(s * rpt, rpt)])

    return deg_kernel


def _ring_propagate(gh, src_v, dst_v, rows_v, acc, gsem, ssem, kf, tail, nb):
    for b in range(nb):
        pltpu.async_copy(gh.at[src_v.at[pl.ds(b * CB, CB)]], rows_v.at[b],
                         gsem)

    def body(j, carry):
        bj = lax.rem(j, nb)
        pltpu.make_async_copy(gh.at[src_v.at[pl.ds(bj * CB, CB)]],
                              rows_v.at[bj], gsem).wait()
        pltpu.async_copy(rows_v.at[bj], acc.at[dst_v.at[pl.ds(j * CB, CB)]],
                         ssem, add=True)
        nxt = j + (nb - 1)

        @pl.when((j >= 1) & (nxt < kf))
        def _():
            bp = lax.rem(nxt, nb)
            pltpu.make_async_copy(rows_v.at[bp],
                                  acc.at[dst_v.at[pl.ds(0, CB)]], ssem).wait()
            pltpu.async_copy(gh.at[src_v.at[pl.ds(nxt * CB, CB)]],
                             rows_v.at[bp], gsem)

        return carry

    lax.fori_loop(0, kf, body, 0)
    for _ in range(nb):
        pltpu.make_async_copy(rows_v.at[0], acc.at[dst_v.at[pl.ds(0, CB)]],
                              ssem).wait()
    if tail:
        t0 = kf * CB
        pltpu.async_copy(gh.at[src_v.at[pl.ds(t0, tail)]],
                         rows_v.at[0, pl.ds(0, tail)], gsem).wait()
        pltpu.sync_copy(rows_v.at[0, pl.ds(0, tail)],
                        acc.at[dst_v.at[pl.ds(t0, tail)]], add=True)


def _make_prop(np_rows, dh, e, nb):
    et = e // NS
    kf = et // CB
    tail = et - kf * CB
    rpt = np_rows // NS
    mesh = plsc.VectorSubcoreMesh(core_axis_name="c", subcore_axis_name="s")

    @functools.partial(
        pl.kernel,
        out_type=jax.ShapeDtypeStruct((NC, np_rows, dh), jnp.float32),
        mesh=mesh,
        compiler_params=pltpu.CompilerParams(use_tc_tiling_on_sc=False),
        scratch_types=[
            pltpu.VMEM((et,), jnp.int32),
            pltpu.VMEM((et,), jnp.int32),
            pltpu.VMEM((nb, CB, dh), jnp.float32),
            pltpu.VMEM_SHARED((np_rows, dh), jnp.float32),
            pltpu.SemaphoreType.DMA,
            pltpu.SemaphoreType.DMA,
        ],
    )
    def prop_kernel(ei_hbm, g_hbm, zeros_hbm, out_hbm,
                    src_v, dst_v, rows_v, acc, gsem, ssem):
        c = lax.axis_index("c")
        s = lax.axis_index("s")
        pltpu.sync_copy(zeros_hbm.at[pl.ds(s * rpt, rpt)],
                        acc.at[pl.ds(s * rpt, rpt)])
        pltpu.sync_copy(ei_hbm.at[0, pl.ds(s * et, et)], src_v)
        pltpu.sync_copy(ei_hbm.at[1, pl.ds(s * et, et)], dst_v)
        plsc.subcore_barrier()
        _ring_propagate(g_hbm.at[c], src_v, dst_v, rows_v, acc,
                        gsem, ssem, kf, tail, nb)
        plsc.subcore_barrier()
        pltpu.sync_copy(acc.at[pl.ds(s * rpt, rpt)],
                        out_hbm.at[c, pl.ds(s * rpt, rpt)])

    return prop_kernel


def _make_prop_final(n, np_rows, dh, e, nb):
    et = e // NS
    kf = et // CB
    tail = et - kf * CB
    rpt = np_rows // NS
    last = NS - 1
    last_cnt = n - last * rpt
    mesh = plsc.VectorSubcoreMesh(core_axis_name="c", subcore_axis_name="s")

    @functools.partial(
        pl.kernel,
        out_type=jax.ShapeDtypeStruct((NC, np_rows, dh), jnp.float32),
        mesh=mesh,
        compiler_params=pltpu.CompilerParams(use_tc_tiling_on_sc=False),
        scratch_types=[
            pltpu.VMEM((et,), jnp.int32),
            pltpu.VMEM((et,), jnp.int32),
            pltpu.VMEM((nb, CB, dh), jnp.float32),
            pltpu.VMEM((rpt, dh), jnp.float32),
            pltpu.VMEM((rpt, dh), jnp.float32),
            pltpu.VMEM((rpt, 16), jnp.float32),
            pltpu.VMEM((dh,), jnp.float32),
            pltpu.VMEM_SHARED((np_rows, dh), jnp.float32),
            pltpu.SemaphoreType.DMA,
            pltpu.SemaphoreType.DMA,
        ],
    )
    def prop_kernel(ei_hbm, g_hbm, zeros_hbm, dinv_hbm, bias_hbm,
                    out_hbm, src_v, dst_v, rows_v, res_v, g2_v, dinv_v,
                    bias_v, acc, gsem, ssem):
        c = lax.axis_index("c")
        s = lax.axis_index("s")
        pltpu.sync_copy(zeros_hbm.at[pl.ds(s * rpt, rpt)],
                        acc.at[pl.ds(s * rpt, rpt)])
        pltpu.sync_copy(ei_hbm.at[0, pl.ds(s * et, et)], src_v)
        pltpu.sync_copy(ei_hbm.at[1, pl.ds(s * et, et)], dst_v)
        pltpu.sync_copy(bias_hbm.at[c], bias_v)
        plsc.subcore_barrier()
        _ring_propagate(g_hbm.at[c], src_v, dst_v, rows_v, acc,
                        gsem, ssem, kf, tail, nb)
        plsc.subcore_barrier()

        r0 = s * rpt
        pltpu.sync_copy(acc.at[pl.ds(r0, rpt)], res_v)
        bias = bias_v[:]

        def fin(i, carry):
            for kk in range(dh // 16):
                sl = pl.ds(kk * 16, 16)
                res_v[i, sl] = (res_v[i, sl] + g2_v[i, sl]) \
                    * dinv_v[i, pl.ds(0, 16)] + bias
            return carry

        @pl.when(s < last)
        def _():
            pltpu.sync_copy(g_hbm.at[c, pl.ds(r0, rpt)], g2_v)
            pltpu.sync_copy(dinv_hbm.at[pl.ds(r0, rpt)], dinv_v)
            lax.fori_loop(0, rpt, fin, 0)

        @pl.when(s == last)
        def _():
            pltpu.sync_copy(g_hbm.at[c, pl.ds(r0, last_cnt)],
                            g2_v.at[pl.ds(0, last_cnt)])
            pltpu.sync_copy(dinv_hbm.at[pl.ds(r0, last_cnt)],
                            dinv_v.at[pl.ds(0, last_cnt)])
            lax.fori_loop(0, last_cnt, fin, 0)

        pltpu.sync_copy(res_v, out_hbm.at[c, pl.ds(r0, rpt)])

    return prop_kernel


def _prescale_body(degp_ref, x_ref, g1_ref, dinv_ref):
    deg = degp_ref[0, :, 0:1] + degp_ref[1, :, 0:1] + 1.0
    dinv = lax.rsqrt(deg)
    d = x_ref.shape[1]
    g1_ref[0] = x_ref[:, : d // 2] * dinv
    g1_ref[1] = x_ref[:, d // 2:] * dinv
    dinv_ref[...] = jnp.broadcast_to(dinv, dinv_ref.shape)


def _mlp_body(dinv16_ref, s1_ref, g1_ref, w1_ref, b1_ref, w2_ref, g2_ref):
    dinv = dinv16_ref[:, 0:1]
    a1 = (jnp.concatenate([s1_ref[0], s1_ref[1]], axis=1)
          + jnp.concatenate([g1_ref[0], g1_ref[1]], axis=1)) * dinv
    h = jnp.dot(a1, w1_ref[...], preferred_element_type=jnp.float32)
    h = jnp.maximum(h + b1_ref[...], 0.0)
    t = jnp.dot(h, w2_ref[...], preferred_element_type=jnp.float32)
    g2 = t * dinv
    ch = t.shape[1] // 2
    g2_ref[0] = g2[:, :ch]
    g2_ref[1] = g2[:, ch:]


def kernel(x, edge_index, W1, b1, W2, b2):
    n, d = x.shape
    h_dim = W1.shape[1]
    c_dim = W2.shape[1]
    e = edge_index.shape[1]
    dh = d // 2

    np_rows = -(-n // (NS * 8)) * NS * 8

    z16 = jnp.zeros((np_rows, 16), jnp.float32)
    zd = jnp.zeros((np_rows, dh), jnp.float32)
    zc = jnp.zeros((np_rows, c_dim // 2), jnp.float32)
    ones = jnp.ones((CB, 16), jnp.float32)

    degp = _make_deg(np_rows, e)(edge_index, z16, ones)

    bn = 1000
    grid = (n // bn,)
    g1, dinv16 = pl.pallas_call(
        _prescale_body,
        grid=grid,
        in_specs=[
            pl.BlockSpec((2, bn, 16), lambda i: (0, i, 0)),
            pl.BlockSpec((bn, d), lambda i: (i, 0)),
        ],
        out_specs=[
            pl.BlockSpec((2, bn, dh), lambda i: (0, i, 0)),
            pl.BlockSpec((bn, 16), lambda i: (i, 0)),
        ],
        out_shape=[
            jax.ShapeDtypeStruct((2, n, dh), jnp.float32),
            jax.ShapeDtypeStruct((n, 16), jnp.float32),
        ],
    )(degp, x)

    s1 = _make_prop(np_rows, dh, e, 4)(edge_index, g1, zd)

    g2 = pl.pallas_call(
        _mlp_body,
        grid=grid,
        in_specs=[
            pl.BlockSpec((bn, 16), lambda i: (i, 0)),
            pl.BlockSpec((2, bn, dh), lambda i: (0, i, 0)),
            pl.BlockSpec((2, bn, dh), lambda i: (0, i, 0)),
            pl.BlockSpec((d, h_dim), lambda i: (0, 0)),
            pl.BlockSpec((1, h_dim), lambda i: (0, 0)),
            pl.BlockSpec((h_dim, c_dim), lambda i: (0, 0)),
        ],
        out_specs=pl.BlockSpec((2, bn, c_dim // 2), lambda i: (0, i, 0)),
        out_shape=jax.ShapeDtypeStruct((2, n, c_dim // 2), jnp.float32),
    )(dinv16, s1, g1, W1, b1.reshape(1, h_dim), W2)

    b2h = b2.reshape(NC, c_dim // 2)
    outh = _make_prop_final(n, np_rows, c_dim // 2, e, 4)(
        edge_index, g2, zc, dinv16, b2h)
    return jnp.concatenate([outh[0, :n], outh[1, :n]], axis=1)


# speedup vs baseline: 1.5592x; 1.1169x over previous
import functools

import jax
import jax.numpy as jnp
from jax import lax
from jax.experimental import pallas as pl
from jax.experimental.pallas import tpu as pltpu
from jax.experimental.pallas import tpu_sc as plsc

NC = 2
NS = 16
NW = NC * NS
CB = 128


def _make_deg(np_rows, e):
    et = e // NW
    kf = et // CB
    tail = et - kf * CB
    rpt = np_rows // NS
    mesh = plsc.VectorSubcoreMesh(core_axis_name="c", subcore_axis_name="s")

    @functools.partial(
        pl.kernel,
        out_type=jax.ShapeDtypeStruct((NC, np_rows, 16), jnp.float32),
        mesh=mesh,
        compiler_params=pltpu.CompilerParams(use_tc_tiling_on_sc=False),
        scratch_types=[
            pltpu.VMEM((et,), jnp.int32),
            pltpu.VMEM((CB, 16), jnp.float32),
            pltpu.VMEM_SHARED((np_rows, 16), jnp.float32),
            pltpu.SemaphoreType.DMA,
        ],
    )
    def deg_kernel(ei_hbm, zeros_hbm, ones_hbm, out_hbm, dst_v, ones_v, acc,
                   sem):
        c = lax.axis_index("c")
        s = lax.axis_index("s")
        wid = c * NS + s
        pltpu.sync_copy(zeros_hbm.at[pl.ds(s * rpt, rpt)],
                        acc.at[pl.ds(s * rpt, rpt)])
        pltpu.sync_copy(ei_hbm.at[1, pl.ds(wid * et, et)], dst_v)
        pltpu.sync_copy(ones_hbm, ones_v)
        plsc.subcore_barrier()

        nbd = 4

        def body(j, carry):
            pltpu.async_copy(ones_v, acc.at[dst_v.at[pl.ds(j * CB, CB)]],
                             sem, add=True)

            @pl.when(j >= nbd)
            def _():
                pltpu.make_async_copy(ones_v,
                                      acc.at[dst_v.at[pl.ds(0, CB)]],
                                      sem).wait()

            return carry

        lax.fori_loop(0, kf, body, 0)
        for _ in range(min(nbd, kf)):
            pltpu.make_async_copy(ones_v, acc.at[dst_v.at[pl.ds(0, CB)]],
                                  sem).wait()
        if tail:
            pltpu.sync_copy(ones_v.at[pl.ds(0, tail)],
                            acc.at[dst_v.at[pl.ds(kf * CB, tail)]], add=True)
        plsc.subcore_barrier()
        pltpu.sync_copy(acc.at[pl.ds(s * rpt, rpt)],
                        out_hbm.at[c, pl.ds(s * rpt, rpt)])

    return deg_kernel


def _ring_propagate(gh, src_v, dst_v, rows_v, acc, gsem, ssem, kf, tail, nb):
    for b in range(nb):
        pltpu.async_copy(gh.at[src_v.at[pl.ds(b * CB, CB)]], rows_v.at[b],
                         gsem)

    def body(j, carry):
        bj = lax.rem(j, nb)
        pltpu.make_async_copy(gh.at[src_v.at[pl.ds(bj * CB, CB)]],
                              rows_v.at[bj], gsem).wait()
        pltpu.async_copy(rows_v.at[bj], acc.at[dst_v.at[pl.ds(j * CB, CB)]],
                         ssem, add=True)
        nxt = j + (nb - 1)

        @pl.when((j >= 1) & (nxt < kf))
        def _():
            bp = lax.rem(nxt, nb)
            pltpu.make_async_copy(rows_v.at[bp],
                                  acc.at[dst_v.at[pl.ds(0, CB)]], ssem).wait()
            pltpu.async_copy(gh.at[src_v.at[pl.ds(nxt * CB, CB)]],
                             rows_v.at[bp], gsem)

        return carry

    lax.fori_loop(0, kf, body, 0)
    for _ in range(nb):
        pltpu.make_async_copy(rows_v.at[0], acc.at[dst_v.at[pl.ds(0, CB)]],
                              ssem).wait()
    if tail:
        t0 = kf * CB
        pltpu.async_copy(gh.at[src_v.at[pl.ds(t0, tail)]],
                         rows_v.at[0, pl.ds(0, tail)], gsem).wait()
        pltpu.sync_copy(rows_v.at[0, pl.ds(0, tail)],
                        acc.at[dst_v.at[pl.ds(t0, tail)]], add=True)


def _make_prop(np_rows, dh, e, nb):
    et = e // NS
    kf = et // CB
    tail = et - kf * CB
    rpt = np_rows // NS
    mesh = plsc.VectorSubcoreMesh(core_axis_name="c", subcore_axis_name="s")

    @functools.partial(
        pl.kernel,
        out_type=jax.ShapeDtypeStruct((NC, np_rows, dh), jnp.float32),
        mesh=mesh,
        compiler_params=pltpu.CompilerParams(use_tc_tiling_on_sc=False),
        scratch_types=[
            pltpu.VMEM((et,), jnp.int32),
            pltpu.VMEM((et,), jnp.int32),
            pltpu.VMEM((nb, CB, dh), jnp.float32),
            pltpu.VMEM_SHARED((np_rows, dh), jnp.float32),
            pltpu.SemaphoreType.DMA,
            pltpu.SemaphoreType.DMA,
        ],
    )
    def prop_kernel(ei_hbm, g_hbm, zeros_hbm, out_hbm,
                    src_v, dst_v, rows_v, acc, gsem, ssem):
        c = lax.axis_index("c")
        s = lax.axis_index("s")
        pltpu.sync_copy(zeros_hbm.at[pl.ds(s * rpt, rpt)],
                        acc.at[pl.ds(s * rpt, rpt)])
        pltpu.sync_copy(ei_hbm.at[0, pl.ds(s * et, et)], src_v)
        pltpu.sync_copy(ei_hbm.at[1, pl.ds(s * et, et)], dst_v)
        plsc.subcore_barrier()
        _ring_propagate(g_hbm.at[c], src_v, dst_v, rows_v, acc,
                        gsem, ssem, kf, tail, nb)
        plsc.subcore_barrier()
        pltpu.sync_copy(acc.at[pl.ds(s * rpt, rpt)],
                        out_hbm.at[c, pl.ds(s * rpt, rpt)])

    return prop_kernel


def _make_prop_final(n, np_rows, dh, e, nb):
    et = e // NS
    kf = et // CB
    tail = et - kf * CB
    rpt = np_rows // NS
    last = NS - 1
    last_cnt = n - last * rpt
    mesh = plsc.VectorSubcoreMesh(core_axis_name="c", subcore_axis_name="s")

    @functools.partial(
        pl.kernel,
        out_type=jax.ShapeDtypeStruct((NC, n, dh), jnp.float32),
        mesh=mesh,
        compiler_params=pltpu.CompilerParams(use_tc_tiling_on_sc=False),
        scratch_types=[
            pltpu.VMEM((et,), jnp.int32),
            pltpu.VMEM((et,), jnp.int32),
            pltpu.VMEM((nb, CB, dh), jnp.float32),
            pltpu.VMEM((rpt, dh), jnp.float32),
            pltpu.VMEM((rpt, dh), jnp.float32),
            pltpu.VMEM((rpt, 16), jnp.float32),
            pltpu.VMEM((dh,), jnp.float32),
            pltpu.VMEM_SHARED((np_rows, dh), jnp.float32),
            pltpu.SemaphoreType.DMA,
            pltpu.SemaphoreType.DMA,
        ],
    )
    def prop_kernel(ei_hbm, g_hbm, zeros_hbm, dinv_hbm, bias_hbm,
                    out_hbm, src_v, dst_v, rows_v, res_v, g2_v, dinv_v,
                    bias_v, acc, gsem, ssem):
        c = lax.axis_index("c")
        s = lax.axis_index("s")
        pltpu.sync_copy(zeros_hbm.at[pl.ds(s * rpt, rpt)],
                        acc.at[pl.ds(s * rpt, rpt)])
        pltpu.sync_copy(ei_hbm.at[0, pl.ds(s * et, et)], src_v)
        pltpu.sync_copy(ei_hbm.at[1, pl.ds(s * et, et)], dst_v)
        pltpu.sync_copy(bias_hbm.at[c], bias_v)
        plsc.subcore_barrier()
        _ring_propagate(g_hbm.at[c], src_v, dst_v, rows_v, acc,
                        gsem, ssem, kf, tail, nb)
        plsc.subcore_barrier()

        r0 = s * rpt
        pltpu.sync_copy(acc.at[pl.ds(r0, rpt)], res_v)
        bias = bias_v[:]

        def fin(i, carry):
            for kk in range(dh // 16):
                sl = pl.ds(kk * 16, 16)
                res_v[i, sl] = (res_v[i, sl] + g2_v[i, sl]) \
                    * dinv_v[i, pl.ds(0, 16)] + bias
            return carry

        @pl.when(s < last)
        def _():
            pltpu.sync_copy(g_hbm.at[c, pl.ds(r0, rpt)], g2_v)
            pltpu.sync_copy(dinv_hbm.at[pl.ds(r0, rpt)], dinv_v)
            lax.fori_loop(0, rpt, fin, 0)
            pltpu.sync_copy(res_v, out_hbm.at[c, pl.ds(r0, rpt)])

        @pl.when(s == last)
        def _():
            pltpu.sync_copy(g_hbm.at[c, pl.ds(r0, last_cnt)],
                            g2_v.at[pl.ds(0, last_cnt)])
            pltpu.sync_copy(dinv_hbm.at[pl.ds(r0, last_cnt)],
                            dinv_v.at[pl.ds(0, last_cnt)])
            lax.fori_loop(0, last_cnt, fin, 0)
            pltpu.sync_copy(res_v.at[pl.ds(0, last_cnt)],
                            out_hbm.at[c, pl.ds(r0, last_cnt)])

    return prop_kernel


def _prescale_body(degp_ref, x_ref, g1_ref, dinv_ref):
    deg = degp_ref[0, :, 0:1] + degp_ref[1, :, 0:1] + 1.0
    dinv = lax.rsqrt(deg)
    d = x_ref.shape[1]
    g1_ref[0] = x_ref[:, : d // 2] * dinv
    g1_ref[1] = x_ref[:, d // 2:] * dinv
    dinv_ref[...] = jnp.broadcast_to(dinv, dinv_ref.shape)


def _mlp_body(dinv16_ref, s1_ref, g1_ref, w1_ref, b1_ref, w2_ref, g2_ref):
    dinv = dinv16_ref[:, 0:1]
    a1 = (jnp.concatenate([s1_ref[0], s1_ref[1]], axis=1)
          + jnp.concatenate([g1_ref[0], g1_ref[1]], axis=1)) * dinv
    h = jnp.dot(a1, w1_ref[...], preferred_element_type=jnp.float32)
    h = jnp.maximum(h + b1_ref[...], 0.0)
    t = jnp.dot(h, w2_ref[...], preferred_element_type=jnp.float32)
    g2 = t * dinv
    ch = t.shape[1] // 2
    g2_ref[0] = g2[:, :ch]
    g2_ref[1] = g2[:, ch:]


def kernel(x, edge_index, W1, b1, W2, b2):
    n, d = x.shape
    h_dim = W1.shape[1]
    c_dim = W2.shape[1]
    e = edge_index.shape[1]
    dh = d // 2

    np_rows = -(-n // (NS * 8)) * NS * 8

    z16 = jnp.zeros((np_rows, 16), jnp.float32)
    zd = jnp.zeros((np_rows, dh), jnp.float32)
    zc = jnp.zeros((np_rows, c_dim // 2), jnp.float32)
    ones = jnp.ones((CB, 16), jnp.float32)

    degp = _make_deg(np_rows, e)(edge_index, z16, ones)

    bn = 1000
    grid = (n // bn,)
    g1, dinv16 = pl.pallas_call(
        _prescale_body,
        grid=grid,
        in_specs=[
            pl.BlockSpec((2, bn, 16), lambda i: (0, i, 0)),
            pl.BlockSpec((bn, d), lambda i: (i, 0)),
        ],
        out_specs=[
            pl.BlockSpec((2, bn, dh), lambda i: (0, i, 0)),
            pl.BlockSpec((bn, 16), lambda i: (i, 0)),
        ],
        out_shape=[
            jax.ShapeDtypeStruct((2, n, dh), jnp.float32),
            jax.ShapeDtypeStruct((n, 16), jnp.float32),
        ],
    )(degp, x)

    s1 = _make_prop(np_rows, dh, e, 5)(edge_index, g1, zd)

    g2 = pl.pallas_call(
        _mlp_body,
        grid=grid,
        in_specs=[
            pl.BlockSpec((bn, 16), lambda i: (i, 0)),
            pl.BlockSpec((2, bn, dh), lambda i: (0, i, 0)),
            pl.BlockSpec((2, bn, dh), lambda i: (0, i, 0)),
            pl.BlockSpec((d, h_dim), lambda i: (0, 0)),
            pl.BlockSpec((1, h_dim), lambda i: (0, 0)),
            pl.BlockSpec((h_dim, c_dim), lambda i: (0, 0)),
        ],
        out_specs=pl.BlockSpec((2, bn, c_dim // 2), lambda i: (0, i, 0)),
        out_shape=jax.ShapeDtypeStruct((2, n, c_dim // 2), jnp.float32),
    )(dinv16, s1, g1, W1, b1.reshape(1, h_dim), W2)

    b2h = b2.reshape(NC, c_dim // 2)
    outh = _make_prop_final(n, np_rows, c_dim // 2, e, 8)(
        edge_index, g2, zc, dinv16, b2h)
    return jnp.concatenate([outh[0], outh[1]], axis=1)
